# baseline (speedup 1.0000x reference)
import functools

import jax
import jax.numpy as jnp
from jax import lax
from jax.experimental import pallas as pl
from jax.experimental.pallas import tpu as pltpu
from jax.experimental.pallas import tpu_sc as plsc

B = 4096
L = 200
D = 64
V = 1000000
V2 = V // 2
NBLK = 3907
V2P = NBLK * 128
NC = 2
NS = 16
NW = NC * NS

_mesh = plsc.VectorSubcoreMesh(core_axis_name="c", subcore_axis_name="s")


def _wid():
    return lax.axis_index("s") * NC + lax.axis_index("c")


def _bcast(x):
    return jnp.full((16,), x, jnp.int32)


@functools.partial(
    pl.kernel,
    out_type=jax.ShapeDtypeStruct((V2P, 128), jnp.float32),
    mesh=_mesh,
    compiler_params=pltpu.CompilerParams(needs_layout_passes=False),
    scratch_types=[
        pltpu.VMEM((2, 64, 128), jnp.float32),
        pltpu.VMEM((128, 128), jnp.float32),
    ],
)
def _transpose_table(tableT_hbm, table2_hbm, in_v, out_v):
    w = _wid()
    iota = lax.iota(jnp.int32, 16)
    ng = (NBLK - 1) // NW + 1

    def block(g, carry):
        blk = w + g * NW

        @pl.when(blk < NBLK)
        def _():
            start = blk * 128
            c0 = start * 2
            c1 = jnp.minimum(c0 + 128, V - D)
            pltpu.sync_copy(tableT_hbm.at[:, pl.ds(c0, 128)], in_v.at[0])
            pltpu.sync_copy(tableT_hbm.at[:, pl.ds(c1, 128)], in_v.at[1])

            def row(j, c2):
                for half in range(2):
                    cr = 2 * j + half
                    plane = _bcast(cr >> 7)
                    col = _bcast(cr & 127)
                    for c16 in range(4):
                        v = plsc.load_gather(
                            in_v, [plane, iota + 16 * c16, col])
                        out_v[j, pl.ds(half * 64 + c16 * 16, 16)] = v
                return c2

            lax.fori_loop(0, 128, row, 0)
            pltpu.sync_copy(out_v, table2_hbm.at[pl.ds(start, 128), :])

        return carry

    lax.fori_loop(0, ng, block, 0)


@functools.partial(
    pl.kernel,
    out_type=jax.ShapeDtypeStruct((L, D, B), jnp.float32),
    mesh=_mesh,
    compiler_params=pltpu.CompilerParams(needs_layout_passes=False),
    scratch_types=[
        pltpu.VMEM((128,), jnp.int32),
        pltpu.VMEM((128,), jnp.int32),
        pltpu.VMEM((128, 128), jnp.float32),
        pltpu.VMEM((D, 128), jnp.float32),
        pltpu.VMEM((L * D,), jnp.float32),
        pltpu.SemaphoreType.DMA,
    ],
)
def _gather_emb(seqT_hbm, table2_hbm, pos_hbm, outT_hbm,
                idx_v, idx2_v, dst_v, obuf, pos_v, sem):
    w = _wid()
    iota = lax.iota(jnp.int32, 16)
    pltpu.sync_copy(pos_hbm, pos_v)

    def unit(l, carry):
        pltpu.sync_copy(seqT_hbm.at[l, pl.ds(w * 128, 128)], idx_v)
        par = []
        for c8 in range(8):
            sl = pl.ds(c8 * 16, 16)
            raw = idx_v[sl]
            idx2_v[sl] = lax.shift_right_logical(raw, 1)
            par.append((raw & 1) * 64)
        pltpu.async_copy(table2_hbm.at[idx2_v], dst_v, sem).wait()

        def col(d, ps):
            pg = plsc.load_gather(pos_v, [_bcast(l * D + d)])
            for c8 in range(8):
                v = plsc.load_gather(
                    dst_v, [iota + 16 * c8, ps[c8] + d])
                obuf[d, pl.ds(c8 * 16, 16)] = v + pg
            return ps

        lax.fori_loop(0, D, col, tuple(par))
        pltpu.sync_copy(obuf, outT_hbm.at[l, :, pl.ds(w * 128, 128)])
        return carry

    lax.fori_loop(0, L, unit, 0)


def kernel(seq, table, pos_table):
    tableT = table.T
    seqT = seq.T.astype(jnp.int32)
    pos_flat = pos_table.reshape(L * D)
    table2 = _transpose_table(tableT)
    outT = _gather_emb(seqT, table2, pos_flat)
    return outT.transpose(2, 0, 1)


# speedup vs baseline: 1.3578x; 1.3578x over previous
import functools

import jax
import jax.numpy as jnp
from jax import lax
from jax.experimental import pallas as pl
from jax.experimental.pallas import tpu as pltpu
from jax.experimental.pallas import tpu_sc as plsc

B = 4096
L = 200
D = 64
V = 1000000
NBLK = 3907
V2P = NBLK * 128
CLAST = V - D
NC = 2
NS = 16
NW = NC * NS
NG = (NBLK - 1) // NW + 1

_mesh = plsc.VectorSubcoreMesh(core_axis_name="c", subcore_axis_name="s")
_params = pltpu.CompilerParams(needs_layout_passes=False)


def _wid():
    return lax.axis_index("s") * NC + lax.axis_index("c")


def _bcast(x):
    return jnp.full((16,), x, jnp.int32)


@functools.partial(
    pl.kernel,
    out_type=jax.ShapeDtypeStruct((V2P, 128), jnp.float32),
    mesh=_mesh,
    compiler_params=_params,
    scratch_types=[
        pltpu.VMEM((2, 2, 64, 128), jnp.float32),
        pltpu.VMEM((2, 128, 128), jnp.float32),
        pltpu.SemaphoreType.DMA,
        pltpu.SemaphoreType.DMA,
        pltpu.SemaphoreType.DMA,
        pltpu.SemaphoreType.DMA,
    ],
)
def _transpose_table(tableT_hbm, table2_hbm, in_v, out_v, si0, si1, so0, so1):
    w = _wid()
    iota = lax.iota(jnp.int32, 16)
    par64 = (iota & 1) * 64
    sin = (si0, si1)
    sout = (so0, so1)

    def srcs(blk):
        c0 = blk * 256
        c1 = jnp.minimum(c0 + 128, CLAST)
        c0 = jnp.minimum(c0, CLAST)
        return c0, c1

    def fire_read(blk, ph):
        c0, c1 = srcs(blk)
        pltpu.async_copy(tableT_hbm.at[:, pl.ds(c0, 128)], in_v.at[ph, 0],
                         sin[ph])
        pltpu.async_copy(tableT_hbm.at[:, pl.ds(c1, 128)], in_v.at[ph, 1],
                         sin[ph])

    def wait_read(blk, ph):
        c0, c1 = srcs(blk)
        pltpu.make_async_copy(tableT_hbm.at[:, pl.ds(c0, 128)],
                              in_v.at[ph, 0], sin[ph]).wait()
        pltpu.make_async_copy(tableT_hbm.at[:, pl.ds(c1, 128)],
                              in_v.at[ph, 1], sin[ph]).wait()

    @pl.when(w < NBLK)
    def _():
        fire_read(w, 0)

    def pair(g, carry):
        for ph in range(2):
            gi = 2 * g + ph
            blk = w + gi * NW
            nblk = blk + NW

            @pl.when(blk < NBLK)
            def _():
                wait_read(blk, ph)

                @pl.when(nblk < NBLK)
                def _():
                    fire_read(nblk, 1 - ph)

                @pl.when(gi >= 2)
                def _():
                    pblk = blk - 2 * NW
                    pltpu.make_async_copy(
                        out_v.at[ph],
                        table2_hbm.at[pl.ds(pblk * 128, 128), :],
                        sout[ph]).wait()

                def colchunk(cc, c2):
                    plane = cc >> 3
                    coff = (cc & 7) * 16
                    rowv = lax.shift_right_logical(cc * 16 + iota, 1)
                    for d in range(D):
                        v = in_v[ph, plane, d, pl.ds(coff, 16)]
                        plsc.store_scatter(out_v.at[ph], [rowv, par64 + d], v)
                    return c2

                lax.fori_loop(0, 16, colchunk, 0)
                pltpu.async_copy(
                    out_v.at[ph],
                    table2_hbm.at[pl.ds(blk * 128, 128), :], sout[ph])

        return carry

    lax.fori_loop(0, (NG + 1) // 2, pair, 0)
    for ph in range(2):
        pltpu.make_async_copy(out_v.at[ph],
                              table2_hbm.at[pl.ds(0, 128), :],
                              sout[ph]).wait()


@functools.partial(
    pl.kernel,
    out_type=jax.ShapeDtypeStruct((L, D, B), jnp.float32),
    mesh=_mesh,
    compiler_params=_params,
    scratch_types=[
        pltpu.VMEM((2, 128), jnp.int32),
        pltpu.VMEM((2, 128), jnp.int32),
        pltpu.VMEM((2, 128, 128), jnp.float32),
        pltpu.VMEM((2, D, 128), jnp.float32),
        pltpu.VMEM((L * D,), jnp.float32),
        pltpu.SemaphoreType.DMA,
        pltpu.SemaphoreType.DMA,
        pltpu.SemaphoreType.DMA,
        pltpu.SemaphoreType.DMA,
    ],
)
def _gather_emb(seqT_hbm, table2_hbm, pos_hbm, outT_hbm,
                idx_v, idx2_v, dst_v, obuf, pos_v, sg0, sg1, sw0, sw1):
    w = _wid()
    iota = lax.iota(jnp.int32, 16)
    sgat = (sg0, sg1)
    swr = (sw0, sw1)
    pltpu.sync_copy(pos_hbm, pos_v)

    def prep_and_fire(l, ph):
        pltpu.sync_copy(seqT_hbm.at[l, pl.ds(w * 128, 128)], idx_v.at[ph])
        for c8 in range(8):
            sl = pl.ds(c8 * 16, 16)
            idx2_v[ph, sl] = lax.shift_right_logical(idx_v[ph, sl], 1)
        pltpu.async_copy(table2_hbm.at[idx2_v.at[ph]], dst_v.at[ph], sgat[ph])

    prep_and_fire(0, 0)

    def pair(g, carry):
        for ph in range(2):
            l = 2 * g + ph
            pltpu.make_async_copy(table2_hbm.at[idx2_v.at[ph]],
                                  dst_v.at[ph], sgat[ph]).wait()

            @pl.when(l + 1 < L)
            def _():
                prep_and_fire(l + 1, 1 - ph)

            @pl.when(l >= 2)
            def _():
                pltpu.make_async_copy(
                    obuf.at[ph],
                    outT_hbm.at[l - 2, :, pl.ds(w * 128, 128)],
                    swr[ph]).wait()

            par = []
            for c8 in range(8):
                par.append((idx_v[ph, pl.ds(c8 * 16, 16)] & 1) * 64)
            lbase = l * D

            def dblk(db, ps):
                base = db * 8
                for dd in range(8):
                    d = base + dd
                    pg = plsc.load_gather(pos_v, [_bcast(lbase + d)])
                    for c8 in range(8):
                        v = plsc.load_gather(
                            dst_v.at[ph], [iota + 16 * c8, ps[c8] + d])
                        obuf[ph, d, pl.ds(c8 * 16, 16)] = v + pg
                return ps

            lax.fori_loop(0, 8, dblk, tuple(par))
            pltpu.async_copy(obuf.at[ph],
                             outT_hbm.at[l, :, pl.ds(w * 128, 128)], swr[ph])
        return carry

    lax.fori_loop(0, L // 2, pair, 0)
    for l in (L - 2, L - 1):
        ph = l % 2
        pltpu.make_async_copy(obuf.at[ph],
                              outT_hbm.at[l, :, pl.ds(w * 128, 128)],
                              swr[ph]).wait()


def kernel(seq, table, pos_table):
    tableT = table.T
    seqT = seq.T.astype(jnp.int32)
    pos_flat = pos_table.reshape(L * D)
    table2 = _transpose_table(tableT)
    outT = _gather_emb(seqT, table2, pos_flat)
    return outT.transpose(2, 0, 1)


# speedup vs baseline: 1.7805x; 1.3114x over previous
import functools

import jax
import jax.numpy as jnp
from jax import lax
from jax.experimental import pallas as pl
from jax.experimental.pallas import tpu as pltpu
from jax.experimental.pallas import tpu_sc as plsc

B = 4096
L = 200
D = 64
V = 1000000
NBLK = 3907
V2P = NBLK * 128
CLAST = V - D
NC = 2
NS = 16
NW = NC * NS
NG = (NBLK - 1) // NW + 1

_mesh = plsc.VectorSubcoreMesh(core_axis_name="c", subcore_axis_name="s")
_params = pltpu.CompilerParams(needs_layout_passes=False)


def _wid():
    return lax.axis_index("s") * NC + lax.axis_index("c")


def _bcast(x):
    return jnp.full((16,), x, jnp.int32)


@functools.partial(
    pl.kernel,
    out_type=jax.ShapeDtypeStruct((V2P, 128), jnp.float32),
    mesh=_mesh,
    compiler_params=_params,
    scratch_types=[
        pltpu.VMEM((2, 2, 64, 128), jnp.float32),
        pltpu.VMEM((2, 128, 128), jnp.float32),
        pltpu.SemaphoreType.DMA,
        pltpu.SemaphoreType.DMA,
        pltpu.SemaphoreType.DMA,
        pltpu.SemaphoreType.DMA,
    ],
)
def _transpose_table(tableT_hbm, table2_hbm, in_v, out_v, si0, si1, so0, so1):
    w = _wid()
    iota = lax.iota(jnp.int32, 16)
    par64 = (iota & 1) * 64
    sin = (si0, si1)
    sout = (so0, so1)

    def srcs(blk):
        c0 = blk * 256
        c1 = jnp.minimum(c0 + 128, CLAST)
        c0 = jnp.minimum(c0, CLAST)
        return c0, c1

    def fire_read(blk, ph):
        c0, c1 = srcs(blk)
        pltpu.async_copy(tableT_hbm.at[:, pl.ds(c0, 128)], in_v.at[ph, 0],
                         sin[ph])
        pltpu.async_copy(tableT_hbm.at[:, pl.ds(c1, 128)], in_v.at[ph, 1],
                         sin[ph])

    def wait_read(blk, ph):
        c0, c1 = srcs(blk)
        pltpu.make_async_copy(tableT_hbm.at[:, pl.ds(c0, 128)],
                              in_v.at[ph, 0], sin[ph]).wait()
        pltpu.make_async_copy(tableT_hbm.at[:, pl.ds(c1, 128)],
                              in_v.at[ph, 1], sin[ph]).wait()

    @pl.when(w < NBLK)
    def _():
        fire_read(w, 0)

    def pair(g, carry):
        for ph in range(2):
            gi = 2 * g + ph
            blk = w + gi * NW
            nblk = blk + NW

            @pl.when(blk < NBLK)
            def _():
                wait_read(blk, ph)

                @pl.when(nblk < NBLK)
                def _():
                    fire_read(nblk, 1 - ph)

                @pl.when(gi >= 2)
                def _():
                    pblk = blk - 2 * NW
                    pltpu.make_async_copy(
                        out_v.at[ph],
                        table2_hbm.at[pl.ds(pblk * 128, 128), :],
                        sout[ph]).wait()

                def colchunk(cc, c2):
                    plane = cc >> 3
                    coff = (cc & 7) * 16
                    rowv = lax.shift_right_logical(cc * 16 + iota, 1)
                    for d0 in range(0, D, 8):
                        vs = [in_v[ph, plane, d0 + k, pl.ds(coff, 16)]
                              for k in range(8)]
                        for k in range(8):
                            plsc.store_scatter(
                                out_v.at[ph], [rowv, par64 + (d0 + k)], vs[k])
                    return c2

                lax.fori_loop(0, 16, colchunk, 0)
                pltpu.async_copy(
                    out_v.at[ph],
                    table2_hbm.at[pl.ds(blk * 128, 128), :], sout[ph])

        return carry

    lax.fori_loop(0, (NG + 1) // 2, pair, 0)
    for ph in range(2):
        pltpu.make_async_copy(out_v.at[ph],
                              table2_hbm.at[pl.ds(0, 128), :],
                              sout[ph]).wait()


@functools.partial(
    pl.kernel,
    out_type=jax.ShapeDtypeStruct((L, D, B), jnp.float32),
    mesh=_mesh,
    compiler_params=_params,
    scratch_types=[
        pltpu.VMEM((2, 128), jnp.int32),
        pltpu.VMEM((2, 128), jnp.int32),
        pltpu.VMEM((2, 128, 128), jnp.float32),
        pltpu.VMEM((2, D, 128), jnp.float32),
        pltpu.VMEM((L * D,), jnp.float32),
        pltpu.SemaphoreType.DMA,
        pltpu.SemaphoreType.DMA,
        pltpu.SemaphoreType.DMA,
        pltpu.SemaphoreType.DMA,
    ],
)
def _gather_emb(seqT_hbm, table2_hbm, pos_hbm, outT_hbm,
                idx_v, idx2_v, dst_v, obuf, pos_v, sg0, sg1, sw0, sw1):
    w = _wid()
    iota = lax.iota(jnp.int32, 16)
    sgat = (sg0, sg1)
    swr = (sw0, sw1)
    pltpu.sync_copy(pos_hbm, pos_v)

    def prep_and_fire(l, ph):
        pltpu.sync_copy(seqT_hbm.at[l, pl.ds(w * 128, 128)], idx_v.at[ph])
        for c8 in range(8):
            sl = pl.ds(c8 * 16, 16)
            idx2_v[ph, sl] = lax.shift_right_logical(idx_v[ph, sl], 1)
        pltpu.async_copy(table2_hbm.at[idx2_v.at[ph]], dst_v.at[ph], sgat[ph])

    prep_and_fire(0, 0)

    def pair(g, carry):
        for ph in range(2):
            l = 2 * g + ph
            pltpu.make_async_copy(table2_hbm.at[idx2_v.at[ph]],
                                  dst_v.at[ph], sgat[ph]).wait()

            @pl.when(l + 1 < L)
            def _():
                prep_and_fire(l + 1, 1 - ph)

            @pl.when(l >= 2)
            def _():
                pltpu.make_async_copy(
                    obuf.at[ph],
                    outT_hbm.at[l - 2, :, pl.ds(w * 128, 128)],
                    swr[ph]).wait()

            rows = []
            cols = []
            for c8 in range(8):
                par = (idx_v[ph, pl.ds(c8 * 16, 16)] & 1) * 64
                rows.append(iota + 16 * c8)
                cols.append(par)
            lbase = l * D

            def dblk(db, carry2):
                rs, cs = carry2
                base = db * 8
                for dd in range(8):
                    d = base + dd
                    pg = plsc.load_gather(pos_v, [_bcast(lbase + d)])
                    vs = [plsc.load_gather(dst_v.at[ph],
                                           [rs[c8], cs[c8] + d])
                          for c8 in range(8)]
                    for c8 in range(8):
                        obuf[ph, d, pl.ds(c8 * 16, 16)] = vs[c8] + pg
                return (rs, cs)

            lax.fori_loop(0, 8, dblk, (tuple(rows), tuple(cols)))
            pltpu.async_copy(obuf.at[ph],
                             outT_hbm.at[l, :, pl.ds(w * 128, 128)], swr[ph])
        return carry

    lax.fori_loop(0, L // 2, pair, 0)
    for l in (L - 2, L - 1):
        ph = l % 2
        pltpu.make_async_copy(obuf.at[ph],
                              outT_hbm.at[l, :, pl.ds(w * 128, 128)],
                              swr[ph]).wait()


def kernel(seq, table, pos_table):
    tableT = table.T
    seqT = seq.T.astype(jnp.int32)
    pos_flat = pos_table.reshape(L * D)
    table2 = _transpose_table(tableT)
    outT = _gather_emb(seqT, table2, pos_flat)
    return outT.transpose(2, 0, 1)


# speedup vs baseline: 2.7428x; 1.5405x over previous
import functools

import jax
import jax.numpy as jnp
from jax import lax
from jax.experimental import pallas as pl
from jax.experimental.pallas import tpu as pltpu
from jax.experimental.pallas import tpu_sc as plsc

B = 4096
L = 200
D = 64
V = 1000000
VH = V // 2
NBLK = 3907
V2P = NBLK * 128
CMAX = 1000064 - 128
NC = 2
NS = 16
NW = NC * NS
NG = (NBLK - 1) // NW + 1

_mesh = plsc.VectorSubcoreMesh(core_axis_name="c", subcore_axis_name="s")
_params = pltpu.CompilerParams(needs_layout_passes=False)


def _wid():
    return lax.axis_index("s") * NC + lax.axis_index("c")


@functools.partial(
    pl.kernel,
    out_type=jax.ShapeDtypeStruct((V2P, 128), jnp.float32),
    mesh=_mesh,
    compiler_params=_params,
    scratch_types=[
        pltpu.VMEM((2, 3, 64, 128), jnp.float32),
        pltpu.VMEM((2, 128, 128), jnp.float32),
        pltpu.SemaphoreType.DMA,
        pltpu.SemaphoreType.DMA,
        pltpu.SemaphoreType.DMA,
        pltpu.SemaphoreType.DMA,
    ],
)
def _transpose_table(tableT_hbm, table2_hbm, in_v, out_v, si0, si1, so0, so1):
    w = _wid()
    iota = lax.iota(jnp.int32, 16)
    diag = [(iota + k) & 15 for k in range(16)]
    sin = (si0, si1)
    sout = (so0, so1)

    def srcs(blk):
        c0 = blk * 128
        c1a = c0 + VH - 32
        c1b = jnp.minimum(c1a + 128, CMAX)
        return c0, c1a, c1b

    def fire_read(blk, ph):
        c0, c1a, c1b = srcs(blk)
        pltpu.async_copy(tableT_hbm.at[:, pl.ds(c0, 128)], in_v.at[ph, 0],
                         sin[ph])
        pltpu.async_copy(tableT_hbm.at[:, pl.ds(c1a, 128)], in_v.at[ph, 1],
                         sin[ph])
        pltpu.async_copy(tableT_hbm.at[:, pl.ds(c1b, 128)], in_v.at[ph, 2],
                         sin[ph])

    def wait_read(blk, ph):
        c0, c1a, c1b = srcs(blk)
        pltpu.make_async_copy(tableT_hbm.at[:, pl.ds(c0, 128)],
                              in_v.at[ph, 0], sin[ph]).wait()
        pltpu.make_async_copy(tableT_hbm.at[:, pl.ds(c1a, 128)],
                              in_v.at[ph, 1], sin[ph]).wait()
        pltpu.make_async_copy(tableT_hbm.at[:, pl.ds(c1b, 128)],
                              in_v.at[ph, 2], sin[ph]).wait()

    @pl.when(w < NBLK)
    def _():
        fire_read(w, 0)

    def pair(g, carry):
        for ph in range(2):
            gi = 2 * g + ph
            blk = w + gi * NW
            nblk = blk + NW

            @pl.when(blk < NBLK)
            def _():
                wait_read(blk, ph)

                @pl.when(nblk < NBLK)
                def _():
                    fire_read(nblk, 1 - ph)

                @pl.when(gi >= 2)
                def _():
                    pblk = blk - 2 * NW
                    pltpu.make_async_copy(
                        out_v.at[ph],
                        table2_hbm.at[pl.ds(pblk * 128, 128), :],
                        sout[ph]).wait()

                def rowblk(r0i, carry2):
                    r0 = r0i * 16
                    dstrow = r0 + iota
                    rsrc = dstrow + 32
                    for h in range(2):
                        if h == 0:
                            pv = dstrow * 0
                            cv = dstrow
                        else:
                            pv = lax.shift_right_logical(rsrc, 7) + 1
                            cv = rsrc & 127
                        for d0 in range(0, D, 16):
                            for k in range(16):
                                dk = d0 + diag[k]
                                v = plsc.load_gather(
                                    in_v.at[ph], [pv, dk, cv])
                                plsc.store_scatter(
                                    out_v.at[ph], [dstrow, h * 64 + dk], v)
                    return carry2

                lax.fori_loop(0, 8, rowblk, 0)
                pltpu.async_copy(
                    out_v.at[ph],
                    table2_hbm.at[pl.ds(blk * 128, 128), :], sout[ph])

        return carry

    lax.fori_loop(0, (NG + 1) // 2, pair, 0)
    for ph in range(2):
        pltpu.make_async_copy(out_v.at[ph],
                              table2_hbm.at[pl.ds(0, 128), :],
                              sout[ph]).wait()


@functools.partial(
    pl.kernel,
    out_type=jax.ShapeDtypeStruct((L, D, B), jnp.float32),
    mesh=_mesh,
    compiler_params=_params,
    scratch_types=[
        pltpu.VMEM((2, 128), jnp.int32),
        pltpu.VMEM((2, 128), jnp.int32),
        pltpu.VMEM((2, 128, 128), jnp.float32),
        pltpu.VMEM((2, D, 128), jnp.float32),
        pltpu.VMEM((L * D,), jnp.float32),
        pltpu.SemaphoreType.DMA,
        pltpu.SemaphoreType.DMA,
        pltpu.SemaphoreType.DMA,
        pltpu.SemaphoreType.DMA,
    ],
)
def _gather_emb(seqT_hbm, table2_hbm, pos_hbm, outT_hbm,
                idx_v, idx2_v, dst_v, obuf, pos_v, sg0, sg1, sw0, sw1):
    w = _wid()
    iota = lax.iota(jnp.int32, 16)
    diag = [(iota + k) & 15 for k in range(16)]
    sgat = (sg0, sg1)
    swr = (sw0, sw1)
    pltpu.sync_copy(pos_hbm, pos_v)

    def prep_and_fire(l, ph):
        pltpu.sync_copy(seqT_hbm.at[l, pl.ds(w * 128, 128)], idx_v.at[ph])
        for c8 in range(8):
            sl = pl.ds(c8 * 16, 16)
            raw = idx_v[ph, sl]
            idx2_v[ph, sl] = jnp.where(raw >= VH, raw - VH, raw)
        pltpu.async_copy(table2_hbm.at[idx2_v.at[ph]], dst_v.at[ph], sgat[ph])

    prep_and_fire(0, 0)

    def pair(g, carry):
        for ph in range(2):
            l = 2 * g + ph
            pltpu.make_async_copy(table2_hbm.at[idx2_v.at[ph]],
                                  dst_v.at[ph], sgat[ph]).wait()

            @pl.when(l + 1 < L)
            def _():
                prep_and_fire(l + 1, 1 - ph)

            @pl.when(l >= 2)
            def _():
                pltpu.make_async_copy(
                    obuf.at[ph],
                    outT_hbm.at[l - 2, :, pl.ds(w * 128, 128)],
                    swr[ph]).wait()

            p64 = []
            bv = []
            for c8 in range(8):
                raw = idx_v[ph, pl.ds(c8 * 16, 16)]
                p64.append(jnp.where(raw >= VH, 64, 0))
                bv.append(iota + 16 * c8)
            lbase = l * D

            def dblk(d0i, carry2):
                ps, bs = carry2
                d0 = d0i * 16
                for k in range(16):
                    dk = d0 + diag[k]
                    pg = plsc.load_gather(pos_v, [lbase + dk])
                    for c8 in range(8):
                        v = plsc.load_gather(
                            dst_v.at[ph], [bs[c8], ps[c8] + dk])
                        plsc.store_scatter(
                            obuf.at[ph], [dk, bs[c8]], v + pg)
                return (ps, bs)

            lax.fori_loop(0, 4, dblk, (tuple(p64), tuple(bv)))
            pltpu.async_copy(obuf.at[ph],
                             outT_hbm.at[l, :, pl.ds(w * 128, 128)], swr[ph])
        return carry

    lax.fori_loop(0, L // 2, pair, 0)
    for l in (L - 2, L - 1):
        ph = l % 2
        pltpu.make_async_copy(obuf.at[ph],
                              outT_hbm.at[l, :, pl.ds(w * 128, 128)],
                              swr[ph]).wait()


def kernel(seq, table, pos_table):
    tableT = table.T
    seqT = seq.T.astype(jnp.int32)
    pos_flat = pos_table.reshape(L * D)
    table2 = _transpose_table(tableT)
    outT = _gather_emb(seqT, table2, pos_flat)
    return outT.transpose(2, 0, 1)


# speedup vs baseline: 5.1447x; 1.8757x over previous
import functools

import jax
import jax.numpy as jnp
from jax import lax
from jax.experimental import pallas as pl
from jax.experimental.pallas import tpu as pltpu
from jax.experimental.pallas import tpu_sc as plsc

B = 4096
L = 200
D = 64
V = 1000000
NBLK = 3907
V2P = NBLK * 128
VH = V2P
CMAX = 1000064 - 128
NC = 2
NS = 16
NW = NC * NS
NG = (NBLK - 1) // NW + 1

_mesh = plsc.VectorSubcoreMesh(core_axis_name="c", subcore_axis_name="s")
_params = pltpu.CompilerParams(needs_layout_passes=False)


def _wid():
    return lax.axis_index("s") * NC + lax.axis_index("c")


@functools.partial(
    pl.kernel,
    out_type=jax.ShapeDtypeStruct((V2P, 128), jnp.float32),
    mesh=_mesh,
    compiler_params=_params,
    scratch_types=[
        pltpu.VMEM((2, 2, 64, 128), jnp.float32),
        pltpu.VMEM((2, 128, 128), jnp.float32),
        pltpu.SemaphoreType.DMA,
        pltpu.SemaphoreType.DMA,
        pltpu.SemaphoreType.DMA,
        pltpu.SemaphoreType.DMA,
    ],
)
def _transpose_table(tableT_hbm, table2_hbm, in_v, out_v, si0, si1, so0, so1):
    w = _wid()
    iota = lax.iota(jnp.int32, 16)
    diag = [(iota + k) & 15 for k in range(16)]
    sin = (si0, si1)
    sout = (so0, so1)

    def srcs(blk):
        c0 = blk * 128
        c1 = jnp.minimum(c0 + VH, CMAX)
        return c0, c1

    def fire_read(blk, ph):
        c0, c1 = srcs(blk)
        pltpu.async_copy(tableT_hbm.at[:, pl.ds(c0, 128)], in_v.at[ph, 0],
                         sin[ph])
        pltpu.async_copy(tableT_hbm.at[:, pl.ds(c1, 128)], in_v.at[ph, 1],
                         sin[ph])

    def wait_read(blk, ph):
        c0, c1 = srcs(blk)
        pltpu.make_async_copy(tableT_hbm.at[:, pl.ds(c0, 128)],
                              in_v.at[ph, 0], sin[ph]).wait()
        pltpu.make_async_copy(tableT_hbm.at[:, pl.ds(c1, 128)],
                              in_v.at[ph, 1], sin[ph]).wait()

    @pl.when(w < NBLK)
    def _():
        fire_read(w, 0)

    def pair(g, carry):
        for ph in range(2):
            gi = 2 * g + ph
            blk = w + gi * NW
            nblk = blk + NW

            @pl.when(blk < NBLK)
            def _():
                wait_read(blk, ph)

                @pl.when(nblk < NBLK)
                def _():
                    fire_read(nblk, 1 - ph)

                @pl.when(gi >= 2)
                def _():
                    pblk = blk - 2 * NW
                    pltpu.make_async_copy(
                        out_v.at[ph],
                        table2_hbm.at[pl.ds(pblk * 128, 128), :],
                        sout[ph]).wait()

                def rowblk(r0i, carry2):
                    r0 = r0i * 16
                    dstrow = r0 + iota
                    groups = [(h, d0) for h in range(2)
                              for d0 in range(0, D, 16)]
                    prev = None
                    for h, d0 in groups:
                        cur = []
                        for k in range(16):
                            dk = d0 + diag[k]
                            cur.append((h * 64 + dk, plsc.load_gather(
                                in_v.at[ph, h], [dk, dstrow])))
                        if prev is not None:
                            for col, v in prev:
                                plsc.store_scatter(
                                    out_v.at[ph], [dstrow, col], v)
                        prev = cur
                    for col, v in prev:
                        plsc.store_scatter(out_v.at[ph], [dstrow, col], v)
                    return carry2

                lax.fori_loop(0, 8, rowblk, 0)
                pltpu.async_copy(
                    out_v.at[ph],
                    table2_hbm.at[pl.ds(blk * 128, 128), :], sout[ph])

        return carry

    lax.fori_loop(0, (NG + 1) // 2, pair, 0)
    for ph in range(2):
        pltpu.make_async_copy(out_v.at[ph],
                              table2_hbm.at[pl.ds(0, 128), :],
                              sout[ph]).wait()


@functools.partial(
    pl.kernel,
    out_type=jax.ShapeDtypeStruct((L, D, B), jnp.float32),
    mesh=_mesh,
    compiler_params=_params,
    scratch_types=[
        pltpu.VMEM((4, 128), jnp.int32),
        pltpu.VMEM((4, 128), jnp.int32),
        pltpu.VMEM((4, 128, 128), jnp.float32),
        pltpu.VMEM((2, D, 128), jnp.float32),
        pltpu.VMEM((L * D,), jnp.float32),
        pltpu.SemaphoreType.DMA,
        pltpu.SemaphoreType.DMA,
        pltpu.SemaphoreType.DMA,
        pltpu.SemaphoreType.DMA,
        pltpu.SemaphoreType.DMA,
        pltpu.SemaphoreType.DMA,
    ],
)
def _gather_emb(seqT_hbm, table2_hbm, pos_hbm, outT_hbm,
                idx_v, idx2_v, dst_v, obuf, pos_v,
                sg0, sg1, sg2, sg3, sw0, sw1):
    w = _wid()
    iota = lax.iota(jnp.int32, 16)
    diag = [(iota + k) & 15 for k in range(16)]
    sgat = (sg0, sg1, sg2, sg3)
    swr = (sw0, sw1)
    pltpu.sync_copy(pos_hbm, pos_v)

    def prep_and_fire(l, ph):
        pltpu.sync_copy(seqT_hbm.at[l, pl.ds(w * 128, 128)], idx_v.at[ph])
        for c8 in range(8):
            sl = pl.ds(c8 * 16, 16)
            raw = idx_v[ph, sl]
            idx2_v[ph, sl] = jnp.where(raw >= VH, raw - VH, raw)
        pltpu.async_copy(table2_hbm.at[idx2_v.at[ph]], dst_v.at[ph], sgat[ph])

    for lp in range(3):
        prep_and_fire(lp, lp)

    def quad(g, carry):
        for ph in range(4):
            l = 4 * g + ph
            wp = ph & 1
            pltpu.make_async_copy(table2_hbm.at[idx2_v.at[ph]],
                                  dst_v.at[ph], sgat[ph]).wait()

            @pl.when(l + 3 < L)
            def _():
                prep_and_fire(l + 3, (ph + 3) % 4)

            @pl.when(l >= 2)
            def _():
                pltpu.make_async_copy(
                    obuf.at[wp],
                    outT_hbm.at[l - 2, :, pl.ds(w * 128, 128)],
                    swr[wp]).wait()

            p64 = []
            bv = []
            for c8 in range(8):
                raw = idx_v[ph, pl.ds(c8 * 16, 16)]
                p64.append(jnp.where(raw >= VH, 64, 0))
                bv.append(iota + 16 * c8)
            lbase = l * D

            def dblk(d0i, carry2):
                ps, bs = carry2
                d0 = d0i * 16
                prev = None
                for k in range(16):
                    dk = d0 + diag[k]
                    pg = plsc.load_gather(pos_v, [lbase + dk])
                    vs = [plsc.load_gather(
                        dst_v.at[ph], [bs[c8], ps[c8] + dk])
                        for c8 in range(8)]
                    if prev is not None:
                        pdk, ppg, pvs = prev
                        for c8 in range(8):
                            plsc.store_scatter(
                                obuf.at[wp], [pdk, bs[c8]], pvs[c8] + ppg)
                    prev = (dk, pg, vs)
                pdk, ppg, pvs = prev
                for c8 in range(8):
                    plsc.store_scatter(
                        obuf.at[wp], [pdk, bs[c8]], pvs[c8] + ppg)
                return (ps, bs)

            lax.fori_loop(0, 4, dblk, (tuple(p64), tuple(bv)))
            pltpu.async_copy(obuf.at[wp],
                             outT_hbm.at[l, :, pl.ds(w * 128, 128)], swr[wp])
        return carry

    lax.fori_loop(0, L // 4, quad, 0)
    for l in (L - 2, L - 1):
        wp = l % 2
        pltpu.make_async_copy(obuf.at[wp],
                              outT_hbm.at[l, :, pl.ds(w * 128, 128)],
                              swr[wp]).wait()


def kernel(seq, table, pos_table):
    tableT = table.T
    seqT = seq.T.astype(jnp.int32)
    pos_flat = pos_table.reshape(L * D)
    table2 = _transpose_table(tableT)
    outT = _gather_emb(seqT, table2, pos_flat)
    return outT.transpose(2, 0, 1)


# speedup vs baseline: 5.6138x; 1.0912x over previous
import functools

import jax
import jax.numpy as jnp
from jax import lax
from jax.experimental import pallas as pl
from jax.experimental.pallas import tpu as pltpu
from jax.experimental.pallas import tpu_sc as plsc

B = 4096
L = 200
D = 64
V = 1000000
NBLK = 3907
V2P = NBLK * 128
VH = V2P
CMAX = 1000064 - 128
NC = 2
NS = 16
NW = NC * NS
NG = (NBLK - 1) // NW + 1

_mesh = plsc.VectorSubcoreMesh(core_axis_name="c", subcore_axis_name="s")
_params = pltpu.CompilerParams(needs_layout_passes=False)


def _wid():
    return lax.axis_index("s") * NC + lax.axis_index("c")


@functools.partial(
    pl.kernel,
    out_type=jax.ShapeDtypeStruct((V2P, 128), jnp.float32),
    mesh=_mesh,
    compiler_params=_params,
    scratch_types=[
        pltpu.VMEM((4, 2, 64, 128), jnp.float32),
        pltpu.VMEM((2, 128, 128), jnp.float32),
        pltpu.SemaphoreType.DMA,
        pltpu.SemaphoreType.DMA,
        pltpu.SemaphoreType.DMA,
        pltpu.SemaphoreType.DMA,
        pltpu.SemaphoreType.DMA,
        pltpu.SemaphoreType.DMA,
    ],
)
def _transpose_table(tableT_hbm, table2_hbm, in_v, out_v,
                     si0, si1, si2, si3, so0, so1):
    w = _wid()
    iota = lax.iota(jnp.int32, 16)
    diag = [(iota + k) & 15 for k in range(16)]
    sin = (si0, si1, si2, si3)
    sout = (so0, so1)

    def srcs(blk):
        c0 = blk * 128
        c1 = jnp.minimum(c0 + VH, CMAX)
        return c0, c1

    def fire_read(blk, ph):
        c0, c1 = srcs(blk)
        pltpu.async_copy(tableT_hbm.at[:, pl.ds(c0, 128)], in_v.at[ph, 0],
                         sin[ph])
        pltpu.async_copy(tableT_hbm.at[:, pl.ds(c1, 128)], in_v.at[ph, 1],
                         sin[ph])

    def wait_read(blk, ph):
        c0, c1 = srcs(blk)
        pltpu.make_async_copy(tableT_hbm.at[:, pl.ds(c0, 128)],
                              in_v.at[ph, 0], sin[ph]).wait()
        pltpu.make_async_copy(tableT_hbm.at[:, pl.ds(c1, 128)],
                              in_v.at[ph, 1], sin[ph]).wait()

    for pf in range(3):
        @pl.when(w + pf * NW < NBLK)
        def _():
            fire_read(w + pf * NW, pf)

    def quadblk(g, carry):
        for ph in range(4):
            gi = 4 * g + ph
            blk = w + gi * NW
            nblk = blk + 3 * NW
            wp = ph & 1

            @pl.when(blk < NBLK)
            def _():
                wait_read(blk, ph)

                @pl.when(nblk < NBLK)
                def _():
                    fire_read(nblk, (ph + 3) % 4)

                @pl.when(gi >= 2)
                def _():
                    pblk = blk - 2 * NW
                    pltpu.make_async_copy(
                        out_v.at[wp],
                        table2_hbm.at[pl.ds(pblk * 128, 128), :],
                        sout[wp]).wait()

                def rowblk(r0i, carry2):
                    r0 = r0i * 16
                    dstrow = r0 + iota
                    groups = [(h, d0) for h in range(2)
                              for d0 in range(0, D, 16)]
                    prev = None
                    for h, d0 in groups:
                        cur = []
                        for k in range(16):
                            dk = d0 + diag[k]
                            cur.append((h * 64 + dk, plsc.load_gather(
                                in_v.at[ph, h], [dk, dstrow])))
                        if prev is not None:
                            for col, v in prev:
                                plsc.store_scatter(
                                    out_v.at[wp], [dstrow, col], v)
                        prev = cur
                    for col, v in prev:
                        plsc.store_scatter(out_v.at[wp], [dstrow, col], v)
                    return carry2

                lax.fori_loop(0, 8, rowblk, 0)
                pltpu.async_copy(
                    out_v.at[wp],
                    table2_hbm.at[pl.ds(blk * 128, 128), :], sout[wp])

        return carry

    lax.fori_loop(0, (NG + 3) // 4, quadblk, 0)
    for ph in range(2):
        pltpu.make_async_copy(out_v.at[ph],
                              table2_hbm.at[pl.ds(0, 128), :],
                              sout[ph]).wait()


@functools.partial(
    pl.kernel,
    out_type=jax.ShapeDtypeStruct((L, D, B), jnp.float32),
    mesh=_mesh,
    compiler_params=_params,
    scratch_types=[
        pltpu.VMEM((4, 128), jnp.int32),
        pltpu.VMEM((4, 128), jnp.int32),
        pltpu.VMEM((4, 128, 128), jnp.float32),
        pltpu.VMEM((2, D, 128), jnp.float32),
        pltpu.VMEM((L * D,), jnp.float32),
        pltpu.SemaphoreType.DMA,
        pltpu.SemaphoreType.DMA,
        pltpu.SemaphoreType.DMA,
        pltpu.SemaphoreType.DMA,
        pltpu.SemaphoreType.DMA,
        pltpu.SemaphoreType.DMA,
    ],
)
def _gather_emb(seqT_hbm, table2_hbm, pos_hbm, outT_hbm,
                idx_v, idx2_v, dst_v, obuf, pos_v,
                sg0, sg1, sg2, sg3, sw0, sw1):
    w = _wid()
    iota = lax.iota(jnp.int32, 16)
    diag = [(iota + k) & 15 for k in range(16)]
    sgat = (sg0, sg1, sg2, sg3)
    swr = (sw0, sw1)
    pltpu.sync_copy(pos_hbm, pos_v)

    def prep_and_fire(l, ph):
        pltpu.sync_copy(seqT_hbm.at[l, pl.ds(w * 128, 128)], idx_v.at[ph])
        for c8 in range(8):
            sl = pl.ds(c8 * 16, 16)
            raw = idx_v[ph, sl]
            idx2_v[ph, sl] = jnp.where(raw >= VH, raw - VH, raw)
        pltpu.async_copy(table2_hbm.at[idx2_v.at[ph]], dst_v.at[ph], sgat[ph])

    for lp in range(3):
        prep_and_fire(lp, lp)

    def quad(g, carry):
        for ph in range(4):
            l = 4 * g + ph
            wp = ph & 1
            pltpu.make_async_copy(table2_hbm.at[idx2_v.at[ph]],
                                  dst_v.at[ph], sgat[ph]).wait()

            @pl.when(l + 3 < L)
            def _():
                prep_and_fire(l + 3, (ph + 3) % 4)

            @pl.when(l >= 2)
            def _():
                pltpu.make_async_copy(
                    obuf.at[wp],
                    outT_hbm.at[l - 2, :, pl.ds(w * 128, 128)],
                    swr[wp]).wait()

            p64 = []
            bv = []
            for c8 in range(8):
                raw = idx_v[ph, pl.ds(c8 * 16, 16)]
                p64.append(jnp.where(raw >= VH, 64, 0))
                bv.append(iota + 16 * c8)
            lbase = l * D

            def dblk(d0i, carry2):
                ps, bs = carry2
                d0 = d0i * 16
                prev = None
                for k in range(16):
                    dk = d0 + diag[k]
                    pg = plsc.load_gather(pos_v, [lbase + dk])
                    vs = [plsc.load_gather(
                        dst_v.at[ph], [bs[c8], ps[c8] + dk])
                        for c8 in range(8)]
                    if prev is not None:
                        pdk, ppg, pvs = prev
                        for c8 in range(8):
                            plsc.store_scatter(
                                obuf.at[wp], [pdk, bs[c8]], pvs[c8] + ppg)
                    prev = (dk, pg, vs)
                pdk, ppg, pvs = prev
                for c8 in range(8):
                    plsc.store_scatter(
                        obuf.at[wp], [pdk, bs[c8]], pvs[c8] + ppg)
                return (ps, bs)

            lax.fori_loop(0, 4, dblk, (tuple(p64), tuple(bv)))
            pltpu.async_copy(obuf.at[wp],
                             outT_hbm.at[l, :, pl.ds(w * 128, 128)], swr[wp])
        return carry

    lax.fori_loop(0, L // 4, quad, 0)
    for l in (L - 2, L - 1):
        wp = l % 2
        pltpu.make_async_copy(obuf.at[wp],
                              outT_hbm.at[l, :, pl.ds(w * 128, 128)],
                              swr[wp]).wait()


def kernel(seq, table, pos_table):
    tableT = table.T
    seqT = seq.T.astype(jnp.int32)
    pos_flat = pos_table.reshape(L * D)
    table2 = _transpose_table(tableT)
    outT = _gather_emb(seqT, table2, pos_flat)
    return outT.transpose(2, 0, 1)
